# Initial kernel scaffold; baseline (speedup 1.0000x reference)
import jax
import jax.numpy as jnp
from jax.experimental import pallas as pl


def kernel(pack_vec, factors, seg_ids, fseg, m1_w1, m1_b1, m1_w2, m1_b2, m1_w3, m1_b3, a1_w, a1_b, m2_w1, m2_b1, m2_w2, m2_b2, m2_w3, m2_b3, m3_w1, m3_b1, m3_w2, m3_b2, m3_w3, m3_b3):
    raise NotImplementedError("write your pallas kernel here")


# speedup vs baseline: 4.9649x; 4.9649x over previous
import jax
import jax.numpy as jnp
from jax.experimental import pallas as pl
from jax.experimental.pallas import tpu as pltpu

_G = 8
_H = 64

_PREC = jax.lax.Precision.DEFAULT


def _stage1_body(x_ref, seg_ref, w1_ref, b1_ref, w2_ref, b2_ref, w3_ref,
                 b3_ref, aw_ref, ab_ref, out_ref):
    x = x_ref[...]
    h = jnp.maximum(
        jnp.dot(x, w1_ref[...], precision=_PREC,
                preferred_element_type=jnp.float32) + b1_ref[...], 0.0)
    h = jnp.maximum(
        jnp.dot(h, w2_ref[...], precision=_PREC,
                preferred_element_type=jnp.float32) + b2_ref[...], 0.0)
    h = jnp.maximum(
        jnp.dot(h, w3_ref[...], precision=_PREC,
                preferred_element_type=jnp.float32) + b3_ref[...], 0.0)
    gate = jax.nn.sigmoid(
        jnp.dot(h, aw_ref[...], precision=_PREC,
                preferred_element_type=jnp.float32) + ab_ref[...])
    contrib = gate * h
    seg = seg_ref[0]
    ids = jax.lax.broadcasted_iota(jnp.int32, (_G, seg.shape[-1]), 0)
    onehot_t = (seg == ids).astype(jnp.float32)
    partial = jnp.dot(onehot_t, contrib, precision=_PREC,
                      preferred_element_type=jnp.float32)

    @pl.when(pl.program_id(1) == 0)
    def _init():
        out_ref[...] = partial[None]

    @pl.when(pl.program_id(1) > 0)
    def _acc():
        out_ref[...] += partial[None]


def _stage2_body(part_ref, fac_ref, fseg_ref,
                 m2w1_ref, m2b1_ref, m2w2_ref, m2b2_ref, m2w3_ref, m2b3_ref,
                 m3w1_ref, m3b1_ref, m3w2_ref, m3b2_ref, m3w3_ref,
                 m3b3_ref, imp_ref, ch_ref):
    f_dim = fac_ref.shape[0]
    pooled = part_ref[0] + part_ref[1]
    gcols = jax.lax.broadcasted_iota(jnp.int32, (f_dim, _G), 1)
    onehot = (fseg_ref[...] == gcols)
    onehot_f = onehot.astype(jnp.float32)
    gathered = jnp.dot(onehot_f, pooled, precision=_PREC,
                       preferred_element_type=jnp.float32)

    fh = fac_ref[...]
    h = jnp.maximum(fh * m2w1_ref[...] + m2b1_ref[...], 0.0)
    h = jnp.maximum(
        jnp.dot(h, m2w2_ref[...], precision=_PREC,
                preferred_element_type=jnp.float32) + m2b2_ref[...], 0.0)
    fvec = jnp.maximum(
        jnp.dot(h, m2w3_ref[...], precision=_PREC,
                preferred_element_type=jnp.float32) + m2b3_ref[...], 0.0)

    fpack = jnp.concatenate([gathered, fvec], axis=1)
    h3 = jnp.maximum(
        jnp.dot(fpack, m3w1_ref[...], precision=_PREC,
                preferred_element_type=jnp.float32) + m3b1_ref[...], 0.0)
    h3 = jnp.maximum(
        jnp.dot(h3, m3w2_ref[...], precision=_PREC,
                preferred_element_type=jnp.float32) + m3b2_ref[...], 0.0)
    l8 = jnp.maximum(
        jnp.dot(h3, m3w3_ref[...], precision=_PREC,
                preferred_element_type=jnp.float32) + m3b3_ref[...], 0.0)

    neg = jnp.float32(-1e30)
    seg_max = jnp.max(jnp.where(onehot, l8, neg), axis=0,
                      keepdims=True)
    exm = jnp.exp(jnp.minimum(l8 - seg_max, 0.0)) * onehot_f
    den = jnp.sum(exm, axis=0, keepdims=True)
    gs = exm / jnp.maximum(den, jnp.float32(1e-30))
    s = jnp.sum(l8 * gs, axis=0, keepdims=True)
    imp_ref[...] = jnp.sum(s, axis=1, keepdims=True)

    gmax = jnp.max(gs, axis=0, keepdims=True)
    rows = jax.lax.broadcasted_iota(jnp.int32, (f_dim, _G), 0)
    pos = jnp.where(onehot & (gs >= gmax), rows, f_dim)
    idx = jnp.min(pos, axis=0, keepdims=True)
    sel = (rows == jnp.minimum(idx, f_dim - 1)).astype(jnp.float32)
    ch_ref[...] = jnp.sum(sel * fh[:, :_G], axis=0, keepdims=True)


def _stage1_call(pack_vec, seg_ids,
                 m1_w1, m1_b1, m1_w2, m1_b2, m1_w3, m1_b3, a1_w, a1_b):
    n = pack_vec.shape[0]
    d_in = pack_vec.shape[1]

    blk = 4096
    while n % (2 * blk) != 0:
        blk //= 2
    nblk = n // blk
    npc = nblk // 2

    seg3 = seg_ids.astype(jnp.int32).reshape(nblk, 1, blk)

    full = lambda shape: pl.BlockSpec(shape, lambda i, j: (0,) * len(shape))
    return pl.pallas_call(
        _stage1_body,
        grid=(2, npc),
        in_specs=[
            pl.BlockSpec((blk, d_in), lambda i, j: (i * npc + j, 0)),
            pl.BlockSpec((1, 1, blk), lambda i, j: (i * npc + j, 0, 0)),
            full((d_in, _H)), full((1, _H)),
            full((_H, _H)), full((1, _H)),
            full((_H, _H)), full((1, _H)),
            full((_H, _H)), full((1, _H)),
        ],
        out_specs=pl.BlockSpec((1, _G, _H), lambda i, j: (i, 0, 0)),
        out_shape=jax.ShapeDtypeStruct((2, _G, _H), jnp.float32),
        compiler_params=pltpu.CompilerParams(
            dimension_semantics=("parallel", "arbitrary")),
    )(pack_vec, seg3,
      m1_w1.T, m1_b1.reshape(1, _H),
      m1_w2.T, m1_b2.reshape(1, _H),
      m1_w3.T, m1_b3.reshape(1, _H),
      a1_w.T, a1_b.reshape(1, _H))


def kernel(pack_vec, factors, seg_ids, fseg,
           m1_w1, m1_b1, m1_w2, m1_b2, m1_w3, m1_b3,
           a1_w, a1_b,
           m2_w1, m2_b1, m2_w2, m2_b2, m2_w3, m2_b3,
           m3_w1, m3_b1, m3_w2, m3_b2, m3_w3, m3_b3):
    f_dim = factors.shape[0]

    part = _stage1_call(pack_vec, seg_ids,
                        m1_w1, m1_b1, m1_w2, m1_b2, m1_w3, m1_b3, a1_w, a1_b)

    full1 = lambda shape: pl.BlockSpec(shape, lambda: (0,) * len(shape))
    imp, ch = pl.pallas_call(
        _stage2_body,
        in_specs=[full1((2, _G, _H)), full1((f_dim, _H)), full1((f_dim, _G)),
                  full1((1, _H)), full1((1, _H)),
                  full1((_H, _H)), full1((1, _H)),
                  full1((_H, _H)), full1((1, _H)),
                  full1((2 * _H, _H)), full1((1, _H)),
                  full1((_H, _H)), full1((1, _H)),
                  full1((_H, _G)), full1((1, _G))],
        out_specs=[full1((1, 1)), full1((1, _G))],
        out_shape=[jax.ShapeDtypeStruct((1, 1), jnp.float32),
                   jax.ShapeDtypeStruct((1, _G), jnp.float32)],
    )(part, jnp.tile(factors, (1, _H)),
      jnp.tile(fseg.astype(jnp.int32).reshape(f_dim, 1), (1, _G)),
      m2_w1.reshape(1, _H), m2_b1.reshape(1, _H),
      m2_w2.T, m2_b2.reshape(1, _H),
      m2_w3.T, m2_b3.reshape(1, _H),
      m3_w1.T, m3_b1.reshape(1, _H),
      m3_w2.T, m3_b2.reshape(1, _H),
      jnp.tile(m3_w3.T, (1, _G)), jnp.tile(m3_b3.reshape(1, 1), (1, _G)))

    return imp[0, 0], ch[0, :]
